# baseline (speedup 1.0000x reference)
import functools

import jax
import jax.numpy as jnp
from jax import lax
from jax.experimental import pallas as pl
from jax.experimental.pallas import tpu as pltpu
from jax.experimental.pallas import tpu_sc as plsc

NC = 2
NS = 16
NW = NC * NS

BUCK = 1568
MAGIC = 85599
MSHIFT = 27
CAP = 28672
K_F = 3200
K_G = 112
K_S = 64


def _leaky(v):
    return jnp.where(v > 0, v, 0.1 * v)


def _wid():
    return lax.axis_index("s") * NC + lax.axis_index("c")


def _filter_body(src_hbm, dst_hbm, ep_hbm, sbufs, dbufs, outp, sems, *, n_edges):
    wid = _wid()
    sent = (wid * BUCK + BUCK) * 65536
    zero16 = jnp.zeros((16,), jnp.int32)
    nchunks = n_edges // K_F

    def pf(i, _):
        outp[pl.ds(i * 16, 16)] = zero16 + sent
        return 0
    lax.fori_loop(0, (CAP + 16) // 16, pf, 0)

    def issue(ci, b):
        pltpu.async_copy(src_hbm.at[pl.ds(ci * K_F, K_F)], sbufs[b], sems[b])
        pltpu.async_copy(dst_hbm.at[pl.ds(ci * K_F, K_F)], dbufs[b], sems[b])

    def drain(b):
        pltpu.make_async_copy(src_hbm.at[pl.ds(0, K_F)], sbufs[b], sems[b]).wait()
        pltpu.make_async_copy(dst_hbm.at[pl.ds(0, K_F)], dbufs[b], sems[b]).wait()

    issue(0, 0)
    issue(1, 1)

    def half(ci, b, offv):
        drain(b)

        def vb(i, offv):
            d = dbufs[b][pl.ds(i * 16, 16)]
            sv = sbufs[b][pl.ds(i * 16, 16)]
            bk = lax.shift_right_logical(d * MAGIC, MSHIFT)
            m = bk == wid
            pos = plsc.cumsum(m.astype(jnp.int32))
            tgt = jnp.minimum(offv + pos - 1, CAP)
            plsc.store_scatter(outp, [tgt], sv + d * 65536, mask=m)
            return offv + plsc.all_reduce_population_count(m)

        offv = lax.fori_loop(0, K_F // 16, vb, offv)

        @pl.when(ci + 2 < nchunks)
        def _():
            issue(ci + 2, b)
        return offv

    def chunk(t2, offv):
        offv = half(t2 * 2, 0, offv)
        return half(t2 * 2 + 1, 1, offv)

    lax.fori_loop(0, nchunks // 2, chunk, jnp.zeros((16,), jnp.int32))
    pltpu.sync_copy(outp.at[pl.ds(0, CAP)], ep_hbm.at[pl.ds(wid * CAP, CAP)])


def _sc_filter(src, dst):
    mesh = plsc.VectorSubcoreMesh(core_axis_name="c", subcore_axis_name="s")
    body = functools.partial(_filter_body, n_edges=src.shape[0])
    f = pl.kernel(
        body,
        out_type=jax.ShapeDtypeStruct((NW * CAP,), jnp.int32),
        mesh=mesh,
        compiler_params=pltpu.CompilerParams(needs_layout_passes=False),
        scratch_types=[[pltpu.VMEM((K_F,), jnp.int32)] * 2,
                       [pltpu.VMEM((K_F,), jnp.int32)] * 2,
                       pltpu.VMEM((CAP + 16,), jnp.int32),
                       [pltpu.SemaphoreType.DMA] * 2],
    )
    return f(src, dst)


def _gather_body(pq_hbm, q_hbm, ep_hbm, g_hbm, epall, sidxs, didxs, bufss,
                 bufds, gbufs, sems, wsems):
    wid = _wid()
    nchunks = CAP // K_G

    pltpu.sync_copy(ep_hbm.at[pl.ds(wid * CAP, CAP)], epall)

    def issue(k, b):
        for i in range(K_G // 16):
            pk = epall[pl.ds(k * K_G + i * 16, 16)]
            sidxs[b][pl.ds(i * 16, 16)] = pk & 65535
            didxs[b][pl.ds(i * 16, 16)] = lax.shift_right_logical(pk, 16)
        q = K_G // 2
        for s4 in range(2):
            sl = pl.ds(s4 * q, q)
            pltpu.async_copy(pq_hbm.at[sidxs[b].at[sl]], bufss[b].at[sl],
                             sems[b])
            pltpu.async_copy(pq_hbm.at[didxs[b].at[sl]], bufds[b].at[sl],
                             sems[b])

    def drain(b):
        pltpu.make_async_copy(pq_hbm.at[pl.ds(0, K_G)], bufss[b], sems[b]).wait()
        pltpu.make_async_copy(pq_hbm.at[pl.ds(0, K_G)], bufds[b], sems[b]).wait()

    issue(0, 0)
    issue(1, 1)

    def half(k, b, first):
        base = wid * CAP + k * K_G
        drain(b)

        @pl.when(jnp.logical_not(first))
        def _():
            pltpu.make_async_copy(g_hbm.at[pl.ds(0, K_G)], gbufs[b],
                                  wsems[b]).wait()
        for r in range(K_G):
            for kk in range(4):
                cs = pl.ds(kk * 16, 16)
                cd = pl.ds(64 + kk * 16, 16)
                gbufs[b][r, cs] = bufss[b][r, cs] + bufds[b][r, cd]
        pltpu.async_copy(gbufs[b], g_hbm.at[pl.ds(base, K_G)], wsems[b])

        @pl.when(k + 2 < nchunks)
        def _():
            issue(k + 2, b)

    def chunk(t2, _):
        half(t2 * 2, 0, t2 == 0)
        half(t2 * 2 + 1, 1, t2 == 0)
        return 0

    lax.fori_loop(0, nchunks // 2, chunk, 0)
    drain2 = pltpu.make_async_copy(g_hbm.at[pl.ds(0, K_G)], gbufs[0], wsems[0])
    drain2.wait()
    pltpu.make_async_copy(g_hbm.at[pl.ds(0, K_G)], gbufs[1], wsems[1]).wait()


def _sc_gather(PQ, Q, edges_p):
    mesh = plsc.VectorSubcoreMesh(core_axis_name="c", subcore_axis_name="s")
    f = pl.kernel(
        _gather_body,
        out_type=jax.ShapeDtypeStruct((NW * CAP, 64), jnp.float32),
        mesh=mesh,
        compiler_params=pltpu.CompilerParams(needs_layout_passes=False),
        scratch_types=[pltpu.VMEM((CAP,), jnp.int32),
                       [pltpu.VMEM((K_G,), jnp.int32)] * 2,
                       [pltpu.VMEM((K_G,), jnp.int32)] * 2,
                       [pltpu.VMEM((K_G, 128), jnp.float32)] * 2,
                       [pltpu.VMEM((K_G, 128), jnp.float32)] * 2,
                       [pltpu.VMEM((K_G, 64), jnp.float32)] * 2,
                       [pltpu.SemaphoreType.DMA] * 2,
                       [pltpu.SemaphoreType.DMA] * 2],
    )
    return f(PQ, Q, edges_p)


def _scatter_body(m_hbm, ep_hbm, agg_hbm, didxs, mbufs, bscr, accf, sems):
    wid = _wid()
    neg_inf = jnp.full((16,), -jnp.inf, jnp.float32)
    iota16 = lax.iota(jnp.int32, 16)
    nchunks = CAP // K_S

    def init(i, _):
        accf[pl.ds(i * 16, 16)] = neg_inf
        return 0
    lax.fori_loop(0, (BUCK + 8) * 64 // 16, init, 0)

    def issue(k, b):
        base = wid * CAP + k * K_S
        pltpu.async_copy(ep_hbm.at[pl.ds(base, K_S)], didxs[b], sems[b])
        pltpu.async_copy(m_hbm.at[pl.ds(base, K_S)], mbufs[b], sems[b])

    def drain(b):
        pltpu.make_async_copy(ep_hbm.at[pl.ds(0, K_S)], didxs[b], sems[b]).wait()
        pltpu.make_async_copy(m_hbm.at[pl.ds(0, K_S)], mbufs[b], sems[b]).wait()

    issue(0, 0)
    issue(1, 1)

    def half(k, b):
        drain(b)
        for i in range(K_S // 16):
            dv = lax.shift_right_logical(didxs[b][pl.ds(i * 16, 16)], 16)
            lvec = dv - wid * BUCK
            bscr[...] = lvec * 64
            for j in range(16):
                bsp = plsc.load_gather(bscr, [jnp.full((16,), j, jnp.int32)])
                idx = bsp + iota16
                for kk in range(4):
                    ik = idx + kk * 16
                    a = plsc.load_gather(accf, [ik])
                    u = mbufs[b][i * 16 + j, pl.ds(kk * 16, 16)]
                    plsc.store_scatter(accf, [ik], jnp.maximum(a, u))

        @pl.when(k + 2 < nchunks)
        def _():
            issue(k + 2, b)

    def chunk(t2, _):
        half(t2 * 2, 0)
        half(t2 * 2 + 1, 1)
        return 0

    lax.fori_loop(0, nchunks // 2, chunk, 0)
    pltpu.sync_copy(accf.at[pl.ds(0, BUCK * 64)],
                    agg_hbm.at[pl.ds(wid * BUCK * 64, BUCK * 64)])


def _sc_scatter_max(M, edges_p, n_pad):
    mesh = plsc.VectorSubcoreMesh(core_axis_name="c", subcore_axis_name="s")
    f = pl.kernel(
        _scatter_body,
        out_type=jax.ShapeDtypeStruct((n_pad * 64,), jnp.float32),
        mesh=mesh,
        compiler_params=pltpu.CompilerParams(needs_layout_passes=False),
        scratch_types=[[pltpu.VMEM((K_S,), jnp.int32)] * 2,
                       [pltpu.VMEM((K_S, 64), jnp.float32)] * 2,
                       pltpu.VMEM((16,), jnp.int32),
                       pltpu.VMEM(((BUCK + 8) * 64,), jnp.float32),
                       [pltpu.SemaphoreType.DMA] * 2],
    )
    return f(M, edges_p)


def _node_stage_kernel(feat_ref, kf_ref, w0f_ref, w0k_ref, b0c_ref,
                       a_ref, b_ref, r_ref, ba_ref,
                       h_ref, pq_ref, q_ref):
    feat = feat_ref[...]
    kf = kf_ref[...]
    h = feat @ w0f_ref[...] + kf * w0k_ref[...] + b0c_ref[...]
    h = jnp.maximum(h, 0.0)
    kr = kf * r_ref[...]
    h_ref[...] = h
    q = h @ b_ref[...] - kr + ba_ref[...]
    pq_ref[...] = jnp.concatenate([h @ a_ref[...] + kr, q], axis=1)
    q_ref[...] = q


def _node_stage(feat, kf, w0f, w0k, b0c, A, B, r, ba, R=2048):
    n = feat.shape[0]
    grid = (n // R,)
    full = lambda s: pl.BlockSpec(s, lambda i: (0, 0))
    row = lambda w: pl.BlockSpec((R, w), lambda i: (i, 0))
    return pl.pallas_call(
        _node_stage_kernel,
        grid=grid,
        in_specs=[row(feat.shape[1]), row(1), full(w0f.shape), full(w0k.shape),
                  full(b0c.shape), full(A.shape), full(B.shape), full(r.shape),
                  full(ba.shape)],
        out_specs=[row(64), row(128), row(64)],
        out_shape=[jax.ShapeDtypeStruct((n, 64), jnp.float32),
                   jax.ShapeDtypeStruct((n, 128), jnp.float32),
                   jax.ShapeDtypeStruct((n, 64), jnp.float32)],
    )(feat, kf, w0f, w0k, b0c, A, B, r, ba)


def _update_stage_kernel(agg_ref, res_ref, kf_ref, a_ref, b_ref, r_ref, ba_ref,
                         x_ref, pq_ref, q_ref):
    agg = agg_ref[...]
    agg = jnp.where(jnp.isfinite(agg), agg, 0.0)
    x = jnp.maximum(agg + res_ref[...], 0.0)
    kr = kf_ref[...] * r_ref[...]
    x_ref[...] = x
    q = x @ b_ref[...] - kr + ba_ref[...]
    pq_ref[...] = jnp.concatenate([x @ a_ref[...] + kr, q], axis=1)
    q_ref[...] = q


def _update_stage(agg, res, kf, A, B, r, ba, R=2048):
    n = agg.shape[0]
    grid = (n // R,)
    full = lambda s: pl.BlockSpec(s, lambda i: (0, 0))
    row = lambda w: pl.BlockSpec((R, w), lambda i: (i, 0))
    return pl.pallas_call(
        _update_stage_kernel,
        grid=grid,
        in_specs=[row(64), row(64), row(1), full(A.shape), full(B.shape),
                  full(r.shape), full(ba.shape)],
        out_specs=[row(64), row(128), row(64)],
        out_shape=[jax.ShapeDtypeStruct((n, 64), jnp.float32),
                   jax.ShapeDtypeStruct((n, 128), jnp.float32),
                   jax.ShapeDtypeStruct((n, 64), jnp.float32)],
    )(agg, res, kf, A, B, r, ba)


def _edge_mm_kernel(g_ref, wb_ref, bb_ref, m_ref):
    m_ref[...] = _leaky(g_ref[...]) @ wb_ref[...] + bb_ref[...]


def _edge_mm(G, Wb, bb, R=4096):
    n = G.shape[0]
    return pl.pallas_call(
        _edge_mm_kernel,
        grid=(n // R,),
        in_specs=[pl.BlockSpec((R, 64), lambda i: (i, 0)),
                  pl.BlockSpec(Wb.shape, lambda i: (0, 0)),
                  pl.BlockSpec(bb.shape, lambda i: (0, 0))],
        out_specs=pl.BlockSpec((R, 64), lambda i: (i, 0)),
        out_shape=jax.ShapeDtypeStruct((n, 64), jnp.float32),
    )(G, Wb, bb)


def _pool_head_kernel(agg2_ref, x1_ref, batch_ref, wf1_ref, bf1_ref,
                      wf2_ref, bf2_ref, out_ref,
                      maxacc, sumacc, cntacc, *, n_valid, R, num_graphs):
    pid = pl.program_id(0)
    nsteps = pl.num_programs(0)

    @pl.when(pid == 0)
    def _init():
        maxacc[...] = jnp.full_like(maxacc, -jnp.inf)
        sumacc[...] = jnp.zeros_like(sumacc)
        cntacc[...] = jnp.zeros_like(cntacc)

    agg2 = agg2_ref[...]
    agg2 = jnp.where(jnp.isfinite(agg2), agg2, 0.0)
    x1 = x1_ref[...]
    x2 = jnp.maximum(agg2 + x1, 0.0)
    cat = jnp.concatenate([x1, x2], axis=1)

    batch = batch_ref[...]
    rowid = jax.lax.broadcasted_iota(jnp.int32, (R, 1), 0)
    valid = (pid * R + rowid) < n_valid

    gid = jax.lax.broadcasted_iota(jnp.int32, (R, num_graphs), 1)
    onehot = jnp.where((batch == gid) & valid, 1.0, 0.0)
    sumacc[...] += jax.lax.dot_general(onehot, cat, (((0,), (0,)), ((), ())))
    cntacc[...] += jnp.sum(onehot, axis=0, keepdims=True)

    g0 = batch[0, 0]
    g1 = batch[R - 1, 0]

    def body(g, _):
        sel = (batch == g) & valid
        vals = jnp.where(sel, cat, -jnp.inf)
        m = jnp.max(vals, axis=0, keepdims=True)
        cur = maxacc[pl.ds(g, 1), :]
        maxacc[pl.ds(g, 1), :] = jnp.maximum(cur, m)
        return 0

    jax.lax.fori_loop(g0, g1 + 1, body, 0)

    @pl.when(pid == nsteps - 1)
    def _final():
        mp = maxacc[...]
        mp = jnp.where(jnp.isfinite(mp), mp, 0.0)
        cnt = jnp.maximum(cntacc[...], 1.0)
        mean = sumacc[...] / cnt.reshape(num_graphs, 1)
        feat = jnp.concatenate([mp, mean], axis=1)
        o = jnp.maximum(feat @ wf1_ref[...] + bf1_ref[...], 0.0)
        o = o @ wf2_ref[...] + bf2_ref[...]
        mx = jnp.max(o, axis=1, keepdims=True)
        lse = jnp.log(jnp.sum(jnp.exp(o - mx), axis=1, keepdims=True)) + mx
        out_ref[...] = o - lse


def _pool_head(agg2, x1, batch2d, Wf1, bf1, Wf2, bf2, n_valid, num_graphs, R=2048):
    n = agg2.shape[0]
    full = lambda s: pl.BlockSpec(s, lambda i: (0, 0))
    row = lambda w: pl.BlockSpec((R, w), lambda i: (i, 0))
    kern = functools.partial(_pool_head_kernel, n_valid=n_valid, R=R,
                             num_graphs=num_graphs)
    return pl.pallas_call(
        kern,
        grid=(n // R,),
        in_specs=[row(64), row(64), row(1), full(Wf1.shape), full(bf1.shape),
                  full(Wf2.shape), full(bf2.shape)],
        out_specs=pl.BlockSpec((num_graphs, 2), lambda i: (0, 0)),
        out_shape=jax.ShapeDtypeStruct((num_graphs, 2), jnp.float32),
        scratch_shapes=[pltpu.VMEM((num_graphs, 128), jnp.float32),
                        pltpu.VMEM((num_graphs, 128), jnp.float32),
                        pltpu.VMEM((1, num_graphs), jnp.float32)],
    )(agg2, x1, batch2d, Wf1, bf1, Wf2, bf2)


def kernel(x, batch, edge_index, W_fe, b_fe, W0, b0, W1a, b1a, W1b, b1b,
           W2a, b2a, W2b, b2b, Wf1, bf1, Wf2, bf2):
    n = x.shape[0]
    H = W0.shape[1]
    feat_dim = x.shape[1] - 1

    R = 2048
    n_pad = ((n + R - 1) // R) * R

    kf = x[:, 0:1]
    feat = x[:, 1:]
    kf_p = jnp.pad(kf, ((0, n_pad - n), (0, 0)))
    feat_p = jnp.pad(feat, ((0, n_pad - n), (0, 0)))
    batch_p = jnp.pad(batch.reshape(n, 1), ((0, n_pad - n), (0, 0)), mode='edge')

    w0f = W0[:feat_dim]
    w0k = W_fe @ W0[feat_dim:]
    b0c = (b_fe @ W0[feat_dim:] + b0).reshape(1, H)

    A1, B1, r1 = W1a[:H], W1a[H:2 * H] - W1a[:H], W1a[2 * H:2 * H + 1]
    A2, B2, r2 = W2a[:H], W2a[H:2 * H] - W2a[:H], W2a[2 * H:2 * H + 1]

    h, PQ1, Q1 = _node_stage(feat_p, kf_p, w0f, w0k, b0c, A1, B1, r1,
                             b1a.reshape(1, H))

    edges_p = _sc_filter(edge_index[0], edge_index[1])

    def conv(PQ, Q, Wb, bb):
        G = _sc_gather(PQ, Q.reshape(n_pad // 2, 128), edges_p)
        M = _edge_mm(G, Wb, bb.reshape(1, H))
        return _sc_scatter_max(M, edges_p, n_pad).reshape(n_pad, 64)

    agg1 = conv(PQ1, Q1, W1b, b1b)
    x1, PQ2, Q2 = _update_stage(agg1, h, kf_p, A2, B2, r2, b2a.reshape(1, H))
    agg2 = conv(PQ2, Q2, W2b, b2b)

    return _pool_head(agg2, x1, batch_p, Wf1, bf1.reshape(1, H),
                      Wf2, bf2.reshape(1, 2), n, 64)


# speedup vs baseline: 1.4170x; 1.4170x over previous
import functools

import jax
import jax.numpy as jnp
from jax import lax
from jax.experimental import pallas as pl
from jax.experimental.pallas import tpu as pltpu
from jax.experimental.pallas import tpu_sc as plsc

NC = 2
NS = 16
NW = NC * NS

BUCK = 1568
MAGIC = 85599
MSHIFT = 27
CAP = 28672
K_F = 3200
K_G = 112
K_S = 64


def _leaky(v):
    return jnp.where(v > 0, v, 0.1 * v)


def _wid():
    return lax.axis_index("s") * NC + lax.axis_index("c")


def _filter_body(src_hbm, dst_hbm, ep_hbm, sbufs, dbufs, outp, sems, *, n_edges):
    wid = _wid()
    sent = (wid * BUCK + BUCK) * 65536
    zero16 = jnp.zeros((16,), jnp.int32)
    nchunks = n_edges // K_F

    def pf(i, _):
        outp[pl.ds(i * 16, 16)] = zero16 + sent
        return 0
    lax.fori_loop(0, (CAP + 16) // 16, pf, 0)

    def issue(ci, b):
        pltpu.async_copy(src_hbm.at[pl.ds(ci * K_F, K_F)], sbufs[b], sems[b])
        pltpu.async_copy(dst_hbm.at[pl.ds(ci * K_F, K_F)], dbufs[b], sems[b])

    def drain(b):
        pltpu.make_async_copy(src_hbm.at[pl.ds(0, K_F)], sbufs[b], sems[b]).wait()
        pltpu.make_async_copy(dst_hbm.at[pl.ds(0, K_F)], dbufs[b], sems[b]).wait()

    issue(0, 0)
    issue(1, 1)

    def half(ci, b, offv):
        drain(b)

        def vb(i, offv):
            d = dbufs[b][pl.ds(i * 16, 16)]
            sv = sbufs[b][pl.ds(i * 16, 16)]
            bk = lax.shift_right_logical(d * MAGIC, MSHIFT)
            m = bk == wid
            pos = plsc.cumsum(m.astype(jnp.int32))
            tgt = jnp.minimum(offv + pos - 1, CAP)
            plsc.store_scatter(outp, [tgt], sv + d * 65536, mask=m)
            return offv + plsc.all_reduce_population_count(m)

        offv = lax.fori_loop(0, K_F // 16, vb, offv)

        @pl.when(ci + 2 < nchunks)
        def _():
            issue(ci + 2, b)
        return offv

    def chunk(t2, offv):
        offv = half(t2 * 2, 0, offv)
        return half(t2 * 2 + 1, 1, offv)

    lax.fori_loop(0, nchunks // 2, chunk, jnp.zeros((16,), jnp.int32))
    pltpu.sync_copy(outp.at[pl.ds(0, CAP)], ep_hbm.at[pl.ds(wid * CAP, CAP)])


def _sc_filter(src, dst):
    mesh = plsc.VectorSubcoreMesh(core_axis_name="c", subcore_axis_name="s")
    body = functools.partial(_filter_body, n_edges=src.shape[0])
    f = pl.kernel(
        body,
        out_type=jax.ShapeDtypeStruct((NW * CAP,), jnp.int32),
        mesh=mesh,
        compiler_params=pltpu.CompilerParams(needs_layout_passes=False),
        scratch_types=[[pltpu.VMEM((K_F,), jnp.int32)] * 2,
                       [pltpu.VMEM((K_F,), jnp.int32)] * 2,
                       pltpu.VMEM((CAP + 16,), jnp.int32),
                       [pltpu.SemaphoreType.DMA] * 2],
    )
    return f(src, dst)


def _gather_body(p_hbm, q_hbm, ep_hbm, g_hbm, epall, sidxs, didxs, bufss,
                 bufds, gbufs, sems, wsems):
    wid = _wid()
    nchunks = CAP // K_G

    pltpu.sync_copy(ep_hbm.at[pl.ds(wid * CAP, CAP)], epall)

    def issue(k, b):
        for i in range(K_G // 16):
            pk = epall[pl.ds(k * K_G + i * 16, 16)]
            sidxs[b][pl.ds(i * 16, 16)] = pk & 65535
            didxs[b][pl.ds(i * 16, 16)] = lax.shift_right_logical(pk, 16)
        pltpu.async_copy(p_hbm.at[sidxs[b]], bufss[b], sems[b])
        pltpu.async_copy(q_hbm.at[didxs[b]], bufds[b], sems[b])

    def drain(b):
        pltpu.make_async_copy(p_hbm.at[pl.ds(0, K_G)], bufss[b], sems[b]).wait()
        pltpu.make_async_copy(q_hbm.at[pl.ds(0, K_G)], bufds[b], sems[b]).wait()

    issue(0, 0)
    issue(1, 1)

    def half(k, b, first):
        base = wid * CAP + k * K_G
        drain(b)

        @pl.when(jnp.logical_not(first))
        def _():
            pltpu.make_async_copy(g_hbm.at[pl.ds(0, K_G)], gbufs[b],
                                  wsems[b]).wait()
        for r in range(K_G):
            for kk in range(4):
                cs = pl.ds(kk * 16, 16)
                gbufs[b][r, cs] = bufss[b][r, cs] + bufds[b][r, cs]
        pltpu.async_copy(gbufs[b], g_hbm.at[pl.ds(base, K_G)], wsems[b])

        @pl.when(k + 2 < nchunks)
        def _():
            issue(k + 2, b)

    def chunk(t2, _):
        half(t2 * 2, 0, t2 == 0)
        half(t2 * 2 + 1, 1, t2 == 0)
        return 0

    lax.fori_loop(0, nchunks // 2, chunk, 0)
    drain2 = pltpu.make_async_copy(g_hbm.at[pl.ds(0, K_G)], gbufs[0], wsems[0])
    drain2.wait()
    pltpu.make_async_copy(g_hbm.at[pl.ds(0, K_G)], gbufs[1], wsems[1]).wait()


def _sc_gather(P, Q, edges_p):
    mesh = plsc.VectorSubcoreMesh(core_axis_name="c", subcore_axis_name="s")
    f = pl.kernel(
        _gather_body,
        out_type=jax.ShapeDtypeStruct((NW * CAP, 64), jnp.float32),
        mesh=mesh,
        compiler_params=pltpu.CompilerParams(needs_layout_passes=False,
                                             use_tc_tiling_on_sc=False),
        scratch_types=[pltpu.VMEM((CAP,), jnp.int32),
                       [pltpu.VMEM((K_G,), jnp.int32)] * 2,
                       [pltpu.VMEM((K_G,), jnp.int32)] * 2,
                       [pltpu.VMEM((K_G, 64), jnp.float32)] * 2,
                       [pltpu.VMEM((K_G, 64), jnp.float32)] * 2,
                       [pltpu.VMEM((K_G, 64), jnp.float32)] * 2,
                       [pltpu.SemaphoreType.DMA] * 2,
                       [pltpu.SemaphoreType.DMA] * 2],
    )
    return f(P, Q, edges_p)


def _scatter_body(m_hbm, ep_hbm, agg_hbm, didxs, mbufs, bscr, accf, sems):
    wid = _wid()
    neg_inf = jnp.full((16,), -jnp.inf, jnp.float32)
    iota16 = lax.iota(jnp.int32, 16)
    nchunks = CAP // K_S

    def init(i, _):
        accf[pl.ds(i * 16, 16)] = neg_inf
        return 0
    lax.fori_loop(0, (BUCK + 8) * 64 // 16, init, 0)

    def issue(k, b):
        base = wid * CAP + k * K_S
        pltpu.async_copy(ep_hbm.at[pl.ds(base, K_S)], didxs[b], sems[b])
        pltpu.async_copy(m_hbm.at[pl.ds(base, K_S)], mbufs[b], sems[b])

    def drain(b):
        pltpu.make_async_copy(ep_hbm.at[pl.ds(0, K_S)], didxs[b], sems[b]).wait()
        pltpu.make_async_copy(m_hbm.at[pl.ds(0, K_S)], mbufs[b], sems[b]).wait()

    issue(0, 0)
    issue(1, 1)

    def half(k, b):
        drain(b)
        for i in range(K_S // 16):
            dv = lax.shift_right_logical(didxs[b][pl.ds(i * 16, 16)], 16)
            lvec = dv - wid * BUCK
            bscr[...] = lvec * 64
            for j in range(16):
                bsp = plsc.load_gather(bscr, [jnp.full((16,), j, jnp.int32)])
                idx = bsp + iota16
                for kk in range(4):
                    ik = idx + kk * 16
                    a = plsc.load_gather(accf, [ik])
                    u = mbufs[b][i * 16 + j, pl.ds(kk * 16, 16)]
                    plsc.store_scatter(accf, [ik], jnp.maximum(a, u))

        @pl.when(k + 2 < nchunks)
        def _():
            issue(k + 2, b)

    def chunk(t2, _):
        half(t2 * 2, 0)
        half(t2 * 2 + 1, 1)
        return 0

    lax.fori_loop(0, nchunks // 2, chunk, 0)
    pltpu.sync_copy(accf.at[pl.ds(0, BUCK * 64)],
                    agg_hbm.at[pl.ds(wid * BUCK * 64, BUCK * 64)])


def _sc_scatter_max(M, edges_p, n_pad):
    mesh = plsc.VectorSubcoreMesh(core_axis_name="c", subcore_axis_name="s")
    f = pl.kernel(
        _scatter_body,
        out_type=jax.ShapeDtypeStruct((n_pad * 64,), jnp.float32),
        mesh=mesh,
        compiler_params=pltpu.CompilerParams(needs_layout_passes=False),
        scratch_types=[[pltpu.VMEM((K_S,), jnp.int32)] * 2,
                       [pltpu.VMEM((K_S, 64), jnp.float32)] * 2,
                       pltpu.VMEM((16,), jnp.int32),
                       pltpu.VMEM(((BUCK + 8) * 64,), jnp.float32),
                       [pltpu.SemaphoreType.DMA] * 2],
    )
    return f(M, edges_p)


def _node_stage_kernel(feat_ref, kf_ref, w0f_ref, w0k_ref, b0c_ref,
                       a_ref, b_ref, r_ref, ba_ref,
                       h_ref, pq_ref, q_ref):
    feat = feat_ref[...]
    kf = kf_ref[...]
    h = feat @ w0f_ref[...] + kf * w0k_ref[...] + b0c_ref[...]
    h = jnp.maximum(h, 0.0)
    kr = kf * r_ref[...]
    h_ref[...] = h
    q = h @ b_ref[...] - kr + ba_ref[...]
    pq_ref[...] = jnp.concatenate([h @ a_ref[...] + kr, q], axis=1)
    q_ref[...] = q


def _node_stage(feat, kf, w0f, w0k, b0c, A, B, r, ba, R=2048):
    n = feat.shape[0]
    grid = (n // R,)
    full = lambda s: pl.BlockSpec(s, lambda i: (0, 0))
    row = lambda w: pl.BlockSpec((R, w), lambda i: (i, 0))
    return pl.pallas_call(
        _node_stage_kernel,
        grid=grid,
        in_specs=[row(feat.shape[1]), row(1), full(w0f.shape), full(w0k.shape),
                  full(b0c.shape), full(A.shape), full(B.shape), full(r.shape),
                  full(ba.shape)],
        out_specs=[row(64), row(128), row(64)],
        out_shape=[jax.ShapeDtypeStruct((n, 64), jnp.float32),
                   jax.ShapeDtypeStruct((n, 128), jnp.float32),
                   jax.ShapeDtypeStruct((n, 64), jnp.float32)],
    )(feat, kf, w0f, w0k, b0c, A, B, r, ba)


def _update_stage_kernel(agg_ref, res_ref, kf_ref, a_ref, b_ref, r_ref, ba_ref,
                         x_ref, pq_ref, q_ref):
    agg = agg_ref[...]
    agg = jnp.where(jnp.isfinite(agg), agg, 0.0)
    x = jnp.maximum(agg + res_ref[...], 0.0)
    kr = kf_ref[...] * r_ref[...]
    x_ref[...] = x
    q = x @ b_ref[...] - kr + ba_ref[...]
    pq_ref[...] = jnp.concatenate([x @ a_ref[...] + kr, q], axis=1)
    q_ref[...] = q


def _update_stage(agg, res, kf, A, B, r, ba, R=2048):
    n = agg.shape[0]
    grid = (n // R,)
    full = lambda s: pl.BlockSpec(s, lambda i: (0, 0))
    row = lambda w: pl.BlockSpec((R, w), lambda i: (i, 0))
    return pl.pallas_call(
        _update_stage_kernel,
        grid=grid,
        in_specs=[row(64), row(64), row(1), full(A.shape), full(B.shape),
                  full(r.shape), full(ba.shape)],
        out_specs=[row(64), row(128), row(64)],
        out_shape=[jax.ShapeDtypeStruct((n, 64), jnp.float32),
                   jax.ShapeDtypeStruct((n, 128), jnp.float32),
                   jax.ShapeDtypeStruct((n, 64), jnp.float32)],
    )(agg, res, kf, A, B, r, ba)


def _edge_mm_kernel(g_ref, wb_ref, bb_ref, m_ref):
    m_ref[...] = _leaky(g_ref[...]) @ wb_ref[...] + bb_ref[...]


def _edge_mm(G, Wb, bb, R=4096):
    n = G.shape[0]
    return pl.pallas_call(
        _edge_mm_kernel,
        grid=(n // R,),
        in_specs=[pl.BlockSpec((R, 64), lambda i: (i, 0)),
                  pl.BlockSpec(Wb.shape, lambda i: (0, 0)),
                  pl.BlockSpec(bb.shape, lambda i: (0, 0))],
        out_specs=pl.BlockSpec((R, 64), lambda i: (i, 0)),
        out_shape=jax.ShapeDtypeStruct((n, 64), jnp.float32),
    )(G, Wb, bb)


def _pool_head_kernel(agg2_ref, x1_ref, batch_ref, wf1_ref, bf1_ref,
                      wf2_ref, bf2_ref, out_ref,
                      maxacc, sumacc, cntacc, *, n_valid, R, num_graphs):
    pid = pl.program_id(0)
    nsteps = pl.num_programs(0)

    @pl.when(pid == 0)
    def _init():
        maxacc[...] = jnp.full_like(maxacc, -jnp.inf)
        sumacc[...] = jnp.zeros_like(sumacc)
        cntacc[...] = jnp.zeros_like(cntacc)

    agg2 = agg2_ref[...]
    agg2 = jnp.where(jnp.isfinite(agg2), agg2, 0.0)
    x1 = x1_ref[...]
    x2 = jnp.maximum(agg2 + x1, 0.0)
    cat = jnp.concatenate([x1, x2], axis=1)

    batch = batch_ref[...]
    rowid = jax.lax.broadcasted_iota(jnp.int32, (R, 1), 0)
    valid = (pid * R + rowid) < n_valid

    gid = jax.lax.broadcasted_iota(jnp.int32, (R, num_graphs), 1)
    onehot = jnp.where((batch == gid) & valid, 1.0, 0.0)
    sumacc[...] += jax.lax.dot_general(onehot, cat, (((0,), (0,)), ((), ())))
    cntacc[...] += jnp.sum(onehot, axis=0, keepdims=True)

    g0 = batch[0, 0]
    g1 = batch[R - 1, 0]

    def body(g, _):
        sel = (batch == g) & valid
        vals = jnp.where(sel, cat, -jnp.inf)
        m = jnp.max(vals, axis=0, keepdims=True)
        cur = maxacc[pl.ds(g, 1), :]
        maxacc[pl.ds(g, 1), :] = jnp.maximum(cur, m)
        return 0

    jax.lax.fori_loop(g0, g1 + 1, body, 0)

    @pl.when(pid == nsteps - 1)
    def _final():
        mp = maxacc[...]
        mp = jnp.where(jnp.isfinite(mp), mp, 0.0)
        cnt = jnp.maximum(cntacc[...], 1.0)
        mean = sumacc[...] / cnt.reshape(num_graphs, 1)
        feat = jnp.concatenate([mp, mean], axis=1)
        o = jnp.maximum(feat @ wf1_ref[...] + bf1_ref[...], 0.0)
        o = o @ wf2_ref[...] + bf2_ref[...]
        mx = jnp.max(o, axis=1, keepdims=True)
        lse = jnp.log(jnp.sum(jnp.exp(o - mx), axis=1, keepdims=True)) + mx
        out_ref[...] = o - lse


def _pool_head(agg2, x1, batch2d, Wf1, bf1, Wf2, bf2, n_valid, num_graphs, R=2048):
    n = agg2.shape[0]
    full = lambda s: pl.BlockSpec(s, lambda i: (0, 0))
    row = lambda w: pl.BlockSpec((R, w), lambda i: (i, 0))
    kern = functools.partial(_pool_head_kernel, n_valid=n_valid, R=R,
                             num_graphs=num_graphs)
    return pl.pallas_call(
        kern,
        grid=(n // R,),
        in_specs=[row(64), row(64), row(1), full(Wf1.shape), full(bf1.shape),
                  full(Wf2.shape), full(bf2.shape)],
        out_specs=pl.BlockSpec((num_graphs, 2), lambda i: (0, 0)),
        out_shape=jax.ShapeDtypeStruct((num_graphs, 2), jnp.float32),
        scratch_shapes=[pltpu.VMEM((num_graphs, 128), jnp.float32),
                        pltpu.VMEM((num_graphs, 128), jnp.float32),
                        pltpu.VMEM((1, num_graphs), jnp.float32)],
    )(agg2, x1, batch2d, Wf1, bf1, Wf2, bf2)


def kernel(x, batch, edge_index, W_fe, b_fe, W0, b0, W1a, b1a, W1b, b1b,
           W2a, b2a, W2b, b2b, Wf1, bf1, Wf2, bf2):
    n = x.shape[0]
    H = W0.shape[1]
    feat_dim = x.shape[1] - 1

    R = 2048
    n_pad = ((n + R - 1) // R) * R

    kf = x[:, 0:1]
    feat = x[:, 1:]
    kf_p = jnp.pad(kf, ((0, n_pad - n), (0, 0)))
    feat_p = jnp.pad(feat, ((0, n_pad - n), (0, 0)))
    batch_p = jnp.pad(batch.reshape(n, 1), ((0, n_pad - n), (0, 0)), mode='edge')

    w0f = W0[:feat_dim]
    w0k = W_fe @ W0[feat_dim:]
    b0c = (b_fe @ W0[feat_dim:] + b0).reshape(1, H)

    A1, B1, r1 = W1a[:H], W1a[H:2 * H] - W1a[:H], W1a[2 * H:2 * H + 1]
    A2, B2, r2 = W2a[:H], W2a[H:2 * H] - W2a[:H], W2a[2 * H:2 * H + 1]

    h, PQ1, Q1 = _node_stage(feat_p, kf_p, w0f, w0k, b0c, A1, B1, r1,
                             b1a.reshape(1, H))

    edges_p = _sc_filter(edge_index[0], edge_index[1])

    def conv(PQ, Q, Wb, bb):
        G = _sc_gather(PQ[:, :64], Q, edges_p)
        M = _edge_mm(G, Wb, bb.reshape(1, H))
        return _sc_scatter_max(M, edges_p, n_pad).reshape(n_pad, 64)

    agg1 = conv(PQ1, Q1, W1b, b1b)
    x1, PQ2, Q2 = _update_stage(agg1, h, kf_p, A2, B2, r2, b2a.reshape(1, H))
    agg2 = conv(PQ2, Q2, W2b, b2b)

    return _pool_head(agg2, x1, batch_p, Wf1, bf1.reshape(1, H),
                      Wf2, bf2.reshape(1, 2), n, 64)
